# Initial kernel scaffold; baseline (speedup 1.0000x reference)
import jax
import jax.numpy as jnp
from jax.experimental import pallas as pl


def kernel(aggregated_vote_xyz, aggregated_vote_features, lang_emb, objectness_scores, tw1, tb1, tw2, tb2, tw3, tb3, lng, lnb, pw1, pb1, pw2, pb2, sw1, sb1, sw2, sb2):
    raise NotImplementedError("write your pallas kernel here")


# speedup vs baseline: 4.9895x; 4.9895x over previous
import functools

import jax
import jax.numpy as jnp
import numpy as np
from jax import lax
from jax.experimental import pallas as pl
from jax.experimental.pallas import tpu as pltpu
from jax.experimental.pallas import tpu_sc as plsc

_B, _N, _K, _LANG = 8, 2048, 16, 256
_STEPS = 3
_PI = 3.141592653589793
_R = 256
_H = lax.Precision.HIGHEST

_NC, _NS = 2, 16
_NW = _NC * _NS
_TOT = _B * _N
_NPT = _TOT // _NW
_CN = 8
_NCHUNK = _NPT // _CN


def _knn_body(xyz_ref, xyzT_ref, objs_ref, gidx_ref, obj_ref):
    b = pl.program_id(0)
    xr = xyz_ref[0]
    xaT = xyzT_ref[0]
    x2r = jnp.sum(xr * xr, axis=1, keepdims=True)
    x2a = jnp.sum(xaT * xaT, axis=0, keepdims=True)
    cross = jnp.dot(xr, xaT)
    d2 = x2r + x2a - 2.0 * cross
    col = lax.broadcasted_iota(jnp.int32, (_R, _N), 1)
    big_i = jnp.int32(2**31 - 1)
    inf = jnp.float32(jnp.inf)
    ids = []
    for j in range(_K + 1):
        m = jnp.min(d2, axis=1, keepdims=True)
        eq = d2 == m
        colsel = jnp.where(eq, col, big_i)
        idxj = jnp.min(colsel, axis=1, keepdims=True)
        if j > 0:
            ids.append(idxj + b * _N)
        d2 = jnp.where(colsel == idxj, inf, d2)
    gidx_ref[0] = jnp.concatenate(ids, axis=1)
    s = objs_ref[0]
    obj_ref[0] = jnp.where(s[:, 1:2] > s[:, 0:1], 1.0, 0.0)


def _run_knn(xyz, xyzT, objs):
    return pl.pallas_call(
        _knn_body,
        grid=(_B, _N // _R),
        in_specs=[
            pl.BlockSpec((1, _R, 3), lambda b, i: (b, i, 0)),
            pl.BlockSpec((1, 3, _N), lambda b, i: (b, 0, 0)),
            pl.BlockSpec((1, _R, 2), lambda b, i: (b, i, 0)),
        ],
        out_specs=[
            pl.BlockSpec((1, _R, _K), lambda b, i: (b, i, 0)),
            pl.BlockSpec((1, _R, 1), lambda b, i: (b, i, 0)),
        ],
        out_shape=[
            jax.ShapeDtypeStruct((_B, _N, _K), jnp.int32),
            jax.ShapeDtypeStruct((_B, _N, 1), jnp.float32),
        ],
    )(xyz, xyzT, objs)


def _theta_body(gidx_ref, xyz_ref, table_ref, lang_ref, tw1_ref, tb1_ref,
                tw2_ref, tb2_ref, tw3_ref, tb3_ref,
                cv_ref, sv_ref, invd_ref):
    b = pl.program_id(0)
    lidx = gidx_ref[0] - b * _N
    xr = xyz_ref[0]
    table = table_ref[0]
    langb = jnp.dot(lang_ref[0], tw1_ref[3:, :]) + tb1_ref[...]
    tw1a = tw1_ref[0:3, :]
    col = lax.broadcasted_iota(jnp.int32, (_R, _N), 1)
    cvs, svs = [], []
    msum = jnp.zeros((_R, 1), jnp.float32)
    for k in range(_K):
        idxk = lidx[:, k:k + 1]
        oh = jnp.where(idxk == col, 1.0, 0.0)
        g = jnp.dot(oh, table, precision=_H)
        delta = g[:, 0:3] - xr
        mask_k = g[:, 3:4]
        h = jax.nn.relu(jnp.dot(delta, tw1a) + langb)
        h = jax.nn.relu(jnp.dot(h, tw2_ref[...]) + tb2_ref[...])
        th = jnp.dot(h, tw3_ref[...]) + tb3_ref[...]
        theta = jnp.tanh(th) * _PI
        cvs.append(mask_k * jnp.cos(theta))
        svs.append(mask_k * jnp.sin(theta))
        msum = msum + mask_k
    invd_ref[0] = 1.0 / jnp.maximum(msum, 1.0)
    cv_ref[0] = jnp.concatenate(cvs, axis=1)
    sv_ref[0] = jnp.concatenate(svs, axis=1)


def _run_theta(gidx, xyz, table, lang, tw1, tb1, tw2, tb2, tw3, tb3):
    wspec = lambda shape: pl.BlockSpec(shape, lambda b, i: tuple(0 for _ in shape))
    return pl.pallas_call(
        _theta_body,
        grid=(_B, _N // _R),
        in_specs=[
            pl.BlockSpec((1, _R, _K), lambda b, i: (b, i, 0)),
            pl.BlockSpec((1, _R, 3), lambda b, i: (b, i, 0)),
            pl.BlockSpec((1, _N, 4), lambda b, i: (b, 0, 0)),
            pl.BlockSpec((1, 1, _LANG), lambda b, i: (b, 0, 0)),
            wspec((_LANG + 3, 128)),
            wspec((1, 128)),
            wspec((128, 128)),
            wspec((1, 128)),
            wspec((128, 1)),
            wspec((1, 1)),
        ],
        out_specs=[
            pl.BlockSpec((1, _R, _K), lambda b, i: (b, i, 0)),
            pl.BlockSpec((1, _R, _K), lambda b, i: (b, i, 0)),
            pl.BlockSpec((1, _R, 1), lambda b, i: (b, i, 0)),
        ],
        out_shape=[
            jax.ShapeDtypeStruct((_B, _N, _K), jnp.float32),
            jax.ShapeDtypeStruct((_B, _N, _K), jnp.float32),
            jax.ShapeDtypeStruct((_B, _N, 1), jnp.float32),
        ],
    )(gidx, xyz, table, lang, tw1, tb1, tw2, tb2, tw3, tb3)


def _sc_step(table, gidxf, cvf, svf, invdf):
    mesh = plsc.VectorSubcoreMesh(core_axis_name="c", subcore_axis_name="s")

    @functools.partial(
        pl.kernel,
        mesh=mesh,
        out_type=jax.ShapeDtypeStruct((_TOT, 128), jnp.float32),
        scratch_types=[
            pltpu.VMEM((_CN * _K,), jnp.int32),
            pltpu.VMEM((_CN * _K, 128), jnp.float32),
            pltpu.VMEM((_CN * _K, 16), jnp.float32),
            pltpu.VMEM((_CN * _K, 16), jnp.float32),
            pltpu.VMEM((_CN, 16), jnp.float32),
            pltpu.VMEM((_CN, 128), jnp.float32),
            pltpu.VMEM((_CN, 128), jnp.float32),
            pltpu.SemaphoreType.DMA,
        ],
    )
    def step(table_h, gidx_h, cv_h, sv_h, invd_h, out_h,
             idx_v, rows_v, cv_v, sv_v, invd_v, uold_v, out_v, sem):
        wid = lax.axis_index("s") * _NC + lax.axis_index("c")
        base_n = wid * _NPT

        def chunk_body(c, carry):
            noff = base_n + c * _CN
            eoff = noff * _K
            pltpu.sync_copy(gidx_h.at[pl.ds(eoff, _CN * _K)], idx_v)
            pltpu.sync_copy(cv_h.at[pl.ds(eoff, _CN * _K)], cv_v)
            pltpu.sync_copy(sv_h.at[pl.ds(eoff, _CN * _K)], sv_v)
            pltpu.sync_copy(invd_h.at[pl.ds(noff, _CN)], invd_v)
            pltpu.sync_copy(table_h.at[pl.ds(noff, _CN)], uold_v)
            pltpu.async_copy(table_h.at[idx_v], rows_v, sem).wait()
            for i in range(_CN):
                def edge(k, accs, i=i):
                    e = i * _K + k
                    cvb = cv_v[e, :]
                    svb = sv_v[e, :]
                    new = list(accs)
                    for j in range(4):
                        xr = rows_v[e, pl.ds(j * 16, 16)]
                        xi = rows_v[e, pl.ds(64 + j * 16, 16)]
                        new[j] = accs[j] + cvb * xr - svb * xi
                        new[4 + j] = accs[4 + j] + svb * xr + cvb * xi
                    return tuple(new)

                acc0 = tuple(jnp.zeros((16,), jnp.float32) for _ in range(8))
                accs = lax.fori_loop(0, _K, edge, acc0)
                invb = invd_v[i, :]
                for j in range(4):
                    out_v[i, pl.ds(j * 16, 16)] = 0.5 * (
                        uold_v[i, pl.ds(j * 16, 16)] + accs[j] * invb)
                    out_v[i, pl.ds(64 + j * 16, 16)] = 0.5 * (
                        uold_v[i, pl.ds(64 + j * 16, 16)] + accs[4 + j] * invb)
            pltpu.sync_copy(out_v, out_h.at[pl.ds(noff, _CN)])
            return carry

        lax.fori_loop(0, _NCHUNK, chunk_body, 0)

    return step(table, gidxf, cvf, svf, invdf)


def _final_body(u_ref, feats_ref, obj_ref, lang_ref, lng_ref, lnb_ref,
                pw1_ref, pb1_ref, pw2_ref, pb2_ref, sw1a_ref, sw1b_ref,
                sw1c_ref, sb1_ref, sw2_ref, sb2_ref, cl_ref):
    u = u_ref[0]
    mu = jnp.mean(u, axis=1, keepdims=True)
    xm = u - mu
    var = jnp.mean(xm * xm, axis=1, keepdims=True)
    normed = xm / jnp.sqrt(var + 1e-5) * lng_ref[...] + lnb_ref[...]
    t1 = jax.nn.relu(jnp.dot(normed, pw1_ref[...]) + pb1_ref[...])
    transported = jnp.dot(t1, pw2_ref[...]) + pb2_ref[...]
    langb = jnp.dot(lang_ref[0], sw1c_ref[...]) + sb1_ref[...]
    z = jax.nn.relu(jnp.dot(transported, sw1a_ref[...])
                    + jnp.dot(feats_ref[0], sw1b_ref[...])
                    + langb)
    conf = jnp.dot(z, sw2_ref[...]) + sb2_ref[...]
    cl_ref[0] = conf * obj_ref[0]


def _run_final(u_d, feats, obj, lang, lng_d, lnb_d, pw1_d, pb1, pw2, pb2,
               sw1a, sw1b, sw1c, sb1, sw2, sb2):
    wspec = lambda shape: pl.BlockSpec(shape, lambda b, i: tuple(0 for _ in shape))
    return pl.pallas_call(
        _final_body,
        grid=(_B, _N // _R),
        in_specs=[
            pl.BlockSpec((1, _R, 128), lambda b, i: (b, i, 0)),
            pl.BlockSpec((1, _R, 128), lambda b, i: (b, i, 0)),
            pl.BlockSpec((1, _R, 1), lambda b, i: (b, i, 0)),
            pl.BlockSpec((1, 1, _LANG), lambda b, i: (b, 0, 0)),
            wspec((1, 128)), wspec((1, 128)),
            wspec((128, 128)), wspec((1, 128)),
            wspec((128, 128)), wspec((1, 128)),
            wspec((128, 128)), wspec((128, 128)),
            wspec((_LANG, 128)), wspec((1, 128)),
            wspec((128, 1)), wspec((1, 1)),
        ],
        out_specs=[pl.BlockSpec((1, _R, 1), lambda b, i: (b, i, 0))],
        out_shape=[jax.ShapeDtypeStruct((_B, _N, 1), jnp.float32)],
    )(u_d, feats, obj, lang, lng_d, lnb_d, pw1_d, pb1, pw2, pb2,
      sw1a, sw1b, sw1c, sb1, sw2, sb2)


def _deint(x):
    s = x.shape[:-1]
    return x.reshape(*s, 64, 2).swapaxes(-1, -2).reshape(*s, 128)


def _reint(x):
    s = x.shape[:-1]
    return x.reshape(*s, 2, 64).swapaxes(-1, -2).reshape(*s, 128)


def kernel(aggregated_vote_xyz, aggregated_vote_features, lang_emb,
           objectness_scores, tw1, tb1, tw2, tb2, tw3, tb3, lng, lnb,
           pw1, pb1, pw2, pb2, sw1, sb1, sw2, sb2):
    xyz = aggregated_vote_xyz
    feats = aggregated_vote_features
    xyzT = jnp.transpose(xyz, (0, 2, 1))

    gidx, obj = _run_knn(xyz, xyzT, objectness_scores)
    lang3 = lang_emb.reshape(_B, 1, _LANG)

    table = jnp.concatenate([xyz, obj], axis=-1)
    cv, sv, invd = _run_theta(
        gidx, xyz, table, lang3, tw1, tb1.reshape(1, 128), tw2,
        tb2.reshape(1, 128), tw3, tb3.reshape(1, 1))

    u = _deint(feats).reshape(_TOT, 128)
    gidxf = gidx.reshape(-1)
    cvf = jnp.broadcast_to(cv.reshape(-1, 1), (_TOT * _K, 16))
    svf = jnp.broadcast_to(sv.reshape(-1, 1), (_TOT * _K, 16))
    invdf = jnp.broadcast_to(invd.reshape(-1, 1), (_TOT, 16))
    for _ in range(_STEPS):
        u = _sc_step(u, gidxf, cvf, svf, invdf)
    u_d = u.reshape(_B, _N, 128)
    updated = _reint(u_d)

    lng_d = _deint(lng).reshape(1, 128)
    lnb_d = _deint(lnb).reshape(1, 128)
    pw1_d = _deint(pw1.T).T
    cl = _run_final(
        u_d, feats, obj, lang3, lng_d, lnb_d, pw1_d,
        pb1.reshape(1, 128), pw2, pb2.reshape(1, 128),
        sw1[:128], sw1[128:256], sw1[256:], sb1.reshape(1, 128),
        sw2, sb2.reshape(1, 1))[0]
    return cl.reshape(_B, _N), updated


# speedup vs baseline: 13.0212x; 2.6097x over previous
import functools

import jax
import jax.numpy as jnp
import numpy as np
from jax import lax
from jax.experimental import pallas as pl
from jax.experimental.pallas import tpu as pltpu
from jax.experimental.pallas import tpu_sc as plsc

_B, _N, _K, _LANG = 8, 2048, 16, 256
_STEPS = 3
_PI = 3.141592653589793
_R = 256
_H = lax.Precision.HIGHEST

_NC, _NS = 2, 16
_NW = _NC * _NS
_TOT = _B * _N
_NPT = _TOT // _NW
_CN = 8
_NCHUNK = _NPT // _CN


def _knn_body(xyz_ref, xyzT_ref, objs_ref, gidx_ref, obj_ref):
    b = pl.program_id(0)
    xr = xyz_ref[0]
    xaT = xyzT_ref[0]
    x2r = jnp.sum(xr * xr, axis=1, keepdims=True)
    x2a = jnp.sum(xaT * xaT, axis=0, keepdims=True)
    cross = jnp.dot(xr, xaT)
    d2 = x2r + x2a - 2.0 * cross
    col = lax.broadcasted_iota(jnp.int32, (_R, _N), 1)
    big_i = jnp.int32(2**31 - 1)
    inf = jnp.float32(jnp.inf)
    ids = []
    for j in range(_K + 1):
        m = jnp.min(d2, axis=1, keepdims=True)
        eq = d2 == m
        colsel = jnp.where(eq, col, big_i)
        idxj = jnp.min(colsel, axis=1, keepdims=True)
        if j > 0:
            ids.append(idxj + b * _N)
        d2 = jnp.where(colsel == idxj, inf, d2)
    gidx_ref[0] = jnp.concatenate(ids, axis=1)
    s = objs_ref[0]
    obj_ref[0] = jnp.where(s[:, 1:2] > s[:, 0:1], 1.0, 0.0)


def _run_knn(xyz, xyzT, objs):
    return pl.pallas_call(
        _knn_body,
        grid=(_B, _N // _R),
        in_specs=[
            pl.BlockSpec((1, _R, 3), lambda b, i: (b, i, 0)),
            pl.BlockSpec((1, 3, _N), lambda b, i: (b, 0, 0)),
            pl.BlockSpec((1, _R, 2), lambda b, i: (b, i, 0)),
        ],
        out_specs=[
            pl.BlockSpec((1, _R, _K), lambda b, i: (b, i, 0)),
            pl.BlockSpec((1, _R, 1), lambda b, i: (b, i, 0)),
        ],
        out_shape=[
            jax.ShapeDtypeStruct((_B, _N, _K), jnp.int32),
            jax.ShapeDtypeStruct((_B, _N, 1), jnp.float32),
        ],
    )(xyz, xyzT, objs)


def _sc_edge_gather(tpad, gidxf):
    mesh = plsc.VectorSubcoreMesh(core_axis_name="c", subcore_axis_name="s")
    epc = _CN * _K

    @functools.partial(
        pl.kernel,
        mesh=mesh,
        out_type=jax.ShapeDtypeStruct((_TOT, 256), jnp.float32),
        scratch_types=[
            pltpu.VMEM((epc,), jnp.int32),
            pltpu.VMEM((epc, 128), jnp.float32),
            pltpu.VMEM((_CN, 128), jnp.float32),
            pltpu.VMEM((_CN, 256), jnp.float32),
            pltpu.SemaphoreType.DMA,
        ],
    )
    def gath(tpad_h, gidx_h, out_h, idx_v, rows_v, xr_v, out_v, sem):
        wid = lax.axis_index("s") * _NC + lax.axis_index("c")
        base_n = wid * _NPT
        lane = lax.iota(jnp.int32, 16)
        lmask = jnp.where(lane < 3, 1.0, 0.0)

        def chunk_body(c, carry):
            noff = base_n + c * _CN
            eoff = noff * _K
            pltpu.sync_copy(gidx_h.at[pl.ds(eoff, epc)], idx_v)
            pltpu.sync_copy(tpad_h.at[pl.ds(noff, _CN)], xr_v)
            pltpu.async_copy(tpad_h.at[idx_v], rows_v, sem).wait()
            for i in range(_CN):
                xrm = xr_v[i, pl.ds(0, 16)] * lmask
                def edge(k, carry2, i=i, xrm=xrm):
                    e = i * _K + k
                    out_v[i, pl.ds(k * 16, 16)] = rows_v[e, pl.ds(0, 16)] - xrm
                    return carry2
                lax.fori_loop(0, _K, edge, 0)
            pltpu.sync_copy(out_v, out_h.at[pl.ds(noff, _CN)])
            return carry

        lax.fori_loop(0, _NCHUNK, chunk_body, 0)

    return gath(tpad, gidxf)


def _theta_body(dm_ref, lang_ref, tw1_ref, tb1_ref,
                tw2_ref, tb2_ref, tw3_ref, tb3_ref,
                cv_ref, sv_ref, invd_ref):
    dm = dm_ref[0]
    langb = jnp.dot(lang_ref[0], tw1_ref[3:, :]) + tb1_ref[...]
    tw1a = tw1_ref[0:3, :]
    ths, ms = [], []
    for k in range(_K):
        delta = dm[:, 16 * k:16 * k + 3]
        ms.append(dm[:, 16 * k + 3:16 * k + 4])
        h = jax.nn.relu(jnp.dot(delta, tw1a) + langb)
        h = jax.nn.relu(jnp.dot(h, tw2_ref[...]) + tb2_ref[...])
        ths.append(jnp.dot(h, tw3_ref[...]) + tb3_ref[...])
    theta = jnp.tanh(jnp.concatenate(ths, axis=1)) * _PI
    mask = jnp.concatenate(ms, axis=1)
    msum = jnp.sum(mask, axis=1, keepdims=True)
    invd_ref[0] = 1.0 / jnp.maximum(msum, 1.0)
    cv_ref[0] = mask * jnp.cos(theta)
    sv_ref[0] = mask * jnp.sin(theta)


def _run_theta(dm, lang, tw1, tb1, tw2, tb2, tw3, tb3):
    wspec = lambda shape: pl.BlockSpec(shape, lambda b, i: tuple(0 for _ in shape))
    return pl.pallas_call(
        _theta_body,
        grid=(_B, _N // _R),
        in_specs=[
            pl.BlockSpec((1, _R, 256), lambda b, i: (b, i, 0)),
            pl.BlockSpec((1, 1, _LANG), lambda b, i: (b, 0, 0)),
            wspec((_LANG + 3, 128)),
            wspec((1, 128)),
            wspec((128, 128)),
            wspec((1, 128)),
            wspec((128, 1)),
            wspec((1, 1)),
        ],
        out_specs=[
            pl.BlockSpec((1, _R, _K), lambda b, i: (b, i, 0)),
            pl.BlockSpec((1, _R, _K), lambda b, i: (b, i, 0)),
            pl.BlockSpec((1, _R, 1), lambda b, i: (b, i, 0)),
        ],
        out_shape=[
            jax.ShapeDtypeStruct((_B, _N, _K), jnp.float32),
            jax.ShapeDtypeStruct((_B, _N, _K), jnp.float32),
            jax.ShapeDtypeStruct((_B, _N, 1), jnp.float32),
        ],
    )(dm, lang, tw1, tb1, tw2, tb2, tw3, tb3)


def _sc_step(table, gidxf, cvf, svf, invdf):
    mesh = plsc.VectorSubcoreMesh(core_axis_name="c", subcore_axis_name="s")

    @functools.partial(
        pl.kernel,
        mesh=mesh,
        out_type=jax.ShapeDtypeStruct((_TOT, 128), jnp.float32),
        scratch_types=[
            pltpu.VMEM((_CN * _K,), jnp.int32),
            pltpu.VMEM((_CN * _K, 128), jnp.float32),
            pltpu.VMEM((_CN * _K, 16), jnp.float32),
            pltpu.VMEM((_CN * _K, 16), jnp.float32),
            pltpu.VMEM((_CN, 16), jnp.float32),
            pltpu.VMEM((_CN, 128), jnp.float32),
            pltpu.VMEM((_CN, 128), jnp.float32),
            pltpu.SemaphoreType.DMA,
        ],
    )
    def step(table_h, gidx_h, cv_h, sv_h, invd_h, out_h,
             idx_v, rows_v, cv_v, sv_v, invd_v, uold_v, out_v, sem):
        wid = lax.axis_index("s") * _NC + lax.axis_index("c")
        base_n = wid * _NPT

        def chunk_body(c, carry):
            noff = base_n + c * _CN
            eoff = noff * _K
            pltpu.sync_copy(gidx_h.at[pl.ds(eoff, _CN * _K)], idx_v)
            pltpu.sync_copy(cv_h.at[pl.ds(eoff, _CN * _K)], cv_v)
            pltpu.sync_copy(sv_h.at[pl.ds(eoff, _CN * _K)], sv_v)
            pltpu.sync_copy(invd_h.at[pl.ds(noff, _CN)], invd_v)
            pltpu.sync_copy(table_h.at[pl.ds(noff, _CN)], uold_v)
            pltpu.async_copy(table_h.at[idx_v], rows_v, sem).wait()
            for i in range(_CN):
                def edge(k, accs, i=i):
                    e = i * _K + k
                    cvb = cv_v[e, :]
                    svb = sv_v[e, :]
                    new = list(accs)
                    for j in range(4):
                        xr = rows_v[e, pl.ds(j * 16, 16)]
                        xi = rows_v[e, pl.ds(64 + j * 16, 16)]
                        new[j] = accs[j] + cvb * xr - svb * xi
                        new[4 + j] = accs[4 + j] + svb * xr + cvb * xi
                    return tuple(new)

                acc0 = tuple(jnp.zeros((16,), jnp.float32) for _ in range(8))
                accs = lax.fori_loop(0, _K, edge, acc0)
                invb = invd_v[i, :]
                for j in range(4):
                    out_v[i, pl.ds(j * 16, 16)] = 0.5 * (
                        uold_v[i, pl.ds(j * 16, 16)] + accs[j] * invb)
                    out_v[i, pl.ds(64 + j * 16, 16)] = 0.5 * (
                        uold_v[i, pl.ds(64 + j * 16, 16)] + accs[4 + j] * invb)
            pltpu.sync_copy(out_v, out_h.at[pl.ds(noff, _CN)])
            return carry

        lax.fori_loop(0, _NCHUNK, chunk_body, 0)

    return step(table, gidxf, cvf, svf, invdf)


def _final_body(u_ref, feats_ref, obj_ref, lang_ref, lng_ref, lnb_ref,
                pw1_ref, pb1_ref, pw2_ref, pb2_ref, sw1a_ref, sw1b_ref,
                sw1c_ref, sb1_ref, sw2_ref, sb2_ref, cl_ref):
    u = u_ref[0]
    mu = jnp.mean(u, axis=1, keepdims=True)
    xm = u - mu
    var = jnp.mean(xm * xm, axis=1, keepdims=True)
    normed = xm / jnp.sqrt(var + 1e-5) * lng_ref[...] + lnb_ref[...]
    t1 = jax.nn.relu(jnp.dot(normed, pw1_ref[...]) + pb1_ref[...])
    transported = jnp.dot(t1, pw2_ref[...]) + pb2_ref[...]
    langb = jnp.dot(lang_ref[0], sw1c_ref[...]) + sb1_ref[...]
    z = jax.nn.relu(jnp.dot(transported, sw1a_ref[...])
                    + jnp.dot(feats_ref[0], sw1b_ref[...])
                    + langb)
    conf = jnp.dot(z, sw2_ref[...]) + sb2_ref[...]
    cl_ref[0] = conf * obj_ref[0]


def _run_final(u_d, feats, obj, lang, lng_d, lnb_d, pw1_d, pb1, pw2, pb2,
               sw1a, sw1b, sw1c, sb1, sw2, sb2):
    wspec = lambda shape: pl.BlockSpec(shape, lambda b, i: tuple(0 for _ in shape))
    return pl.pallas_call(
        _final_body,
        grid=(_B, _N // _R),
        in_specs=[
            pl.BlockSpec((1, _R, 128), lambda b, i: (b, i, 0)),
            pl.BlockSpec((1, _R, 128), lambda b, i: (b, i, 0)),
            pl.BlockSpec((1, _R, 1), lambda b, i: (b, i, 0)),
            pl.BlockSpec((1, 1, _LANG), lambda b, i: (b, 0, 0)),
            wspec((1, 128)), wspec((1, 128)),
            wspec((128, 128)), wspec((1, 128)),
            wspec((128, 128)), wspec((1, 128)),
            wspec((128, 128)), wspec((128, 128)),
            wspec((_LANG, 128)), wspec((1, 128)),
            wspec((128, 1)), wspec((1, 1)),
        ],
        out_specs=[pl.BlockSpec((1, _R, 1), lambda b, i: (b, i, 0))],
        out_shape=[jax.ShapeDtypeStruct((_B, _N, 1), jnp.float32)],
    )(u_d, feats, obj, lang, lng_d, lnb_d, pw1_d, pb1, pw2, pb2,
      sw1a, sw1b, sw1c, sb1, sw2, sb2)


def _deint(x):
    s = x.shape[:-1]
    return x.reshape(*s, 64, 2).swapaxes(-1, -2).reshape(*s, 128)


def _reint(x):
    s = x.shape[:-1]
    return x.reshape(*s, 2, 64).swapaxes(-1, -2).reshape(*s, 128)


def kernel(aggregated_vote_xyz, aggregated_vote_features, lang_emb,
           objectness_scores, tw1, tb1, tw2, tb2, tw3, tb3, lng, lnb,
           pw1, pb1, pw2, pb2, sw1, sb1, sw2, sb2):
    xyz = aggregated_vote_xyz
    feats = aggregated_vote_features
    xyzT = jnp.transpose(xyz, (0, 2, 1))

    gidx, obj = _run_knn(xyz, xyzT, objectness_scores)
    lang3 = lang_emb.reshape(_B, 1, _LANG)
    gidxf = gidx.reshape(-1)

    tpad = jnp.concatenate(
        [xyz, obj, jnp.zeros((_B, _N, 124), jnp.float32)], axis=-1)
    dm = _sc_edge_gather(tpad.reshape(_TOT, 128), gidxf)
    cv, sv, invd = _run_theta(
        dm.reshape(_B, _N, 256), lang3, tw1, tb1.reshape(1, 128), tw2,
        tb2.reshape(1, 128), tw3, tb3.reshape(1, 1))

    u = _deint(feats).reshape(_TOT, 128)
    cvf = jnp.broadcast_to(cv.reshape(-1, 1), (_TOT * _K, 16))
    svf = jnp.broadcast_to(sv.reshape(-1, 1), (_TOT * _K, 16))
    invdf = jnp.broadcast_to(invd.reshape(-1, 1), (_TOT, 16))
    for _ in range(_STEPS):
        u = _sc_step(u, gidxf, cvf, svf, invdf)
    u_d = u.reshape(_B, _N, 128)
    updated = _reint(u_d)

    lng_d = _deint(lng).reshape(1, 128)
    lnb_d = _deint(lnb).reshape(1, 128)
    pw1_d = _deint(pw1.T).T
    cl = _run_final(
        u_d, feats, obj, lang3, lng_d, lnb_d, pw1_d,
        pb1.reshape(1, 128), pw2, pb2.reshape(1, 128),
        sw1[:128], sw1[128:256], sw1[256:], sb1.reshape(1, 128),
        sw2, sb2.reshape(1, 1))[0]
    return cl.reshape(_B, _N), updated
